# baseline (speedup 1.0000x reference)
import functools

import jax
import jax.numpy as jnp
from jax import lax
from jax.experimental import pallas as pl
from jax.experimental.pallas import tpu as pltpu
from jax.experimental.pallas import tpu_sc as plsc

_H = 128
_NG = 16
_ROW = 2000
_EROW = 2048
_NS = 16


def _silu(x):
    return x / (1.0 + jnp.exp(-x))


def _racc_for(nacc):
    return -(-(nacc + 1) // 2048) * 2048


def _eb_for(nacc):
    return 128 if nacc <= 16000 else 64


def _emb_body(x_ref, w_ref, b_ref, o_ref):
    o_ref[...] = jnp.dot(x_ref[...], w_ref[...],
                         preferred_element_type=jnp.float32) + b_ref[...]


@functools.lru_cache(maxsize=None)
def _emb_call(n):
    return pl.pallas_call(
        _emb_body,
        grid=(n // _ROW,),
        in_specs=[
            pl.BlockSpec((_ROW, _H), lambda i: (i, 0)),
            pl.BlockSpec((_H, _H), lambda i: (0, 0)),
            pl.BlockSpec((1, _H), lambda i: (0, 0)),
        ],
        out_specs=pl.BlockSpec((_ROW, _H), lambda i: (i, 0)),
        out_shape=jax.ShapeDtypeStruct((n, _H), jnp.float32),
    )


def _proj_body(nproj, x_ref, w_ref, *o_refs):
    y = jnp.dot(x_ref[...], w_ref[0], preferred_element_type=jnp.float32)
    for p in range(nproj):
        o_refs[p][...] = y[:, p * 64:(p + 1) * 64]


@functools.lru_cache(maxsize=None)
def _proj_call(n, nproj):
    nb = n // _ROW
    return pl.pallas_call(
        functools.partial(_proj_body, nproj),
        grid=(nb, 2),
        in_specs=[
            pl.BlockSpec((_ROW, _H), lambda i, h: (i, 0)),
            pl.BlockSpec((1, _H, nproj * 64), lambda i, h: (h, 0, 0)),
        ],
        out_specs=[pl.BlockSpec((_ROW, 64), lambda i, h, nb=nb: (h * nb + i, 0))
                   for _ in range(nproj)],
        out_shape=[jax.ShapeDtypeStruct((2 * n, 64), jnp.float32)
                   for _ in range(nproj)],
    )


def _cprep_body(inv_ref, w_ref, b_ref, o_ref):
    o_ref[...] = jnp.dot(inv_ref[...], w_ref[0],
                         preferred_element_type=jnp.float32) + b_ref[0]


@functools.lru_cache(maxsize=None)
def _cprep_call(e, epad):
    nb = epad // _EROW
    return pl.pallas_call(
        _cprep_body,
        grid=(nb, 2),
        in_specs=[
            pl.BlockSpec((_EROW, 3), lambda i, h: (i, 0)),
            pl.BlockSpec((1, 3, 64), lambda i, h: (h, 0, 0)),
            pl.BlockSpec((1, 1, 64), lambda i, h: (h, 0, 0)),
        ],
        out_specs=pl.BlockSpec((_EROW, 64), lambda i, h, nb=nb: (h * nb + i, 0)),
        out_shape=jax.ShapeDtypeStruct((2 * epad, 64), jnp.float32),
    )


def _upd_body(has_inc, *refs):
    if has_inc:
        (x_ref, ha_ref, da_ref, hi_ref, di_ref, u1a_ref, ma_ref, mi_ref,
         cab_ref, u2_ref, bu2_ref, o_ref) = refs
    else:
        (x_ref, ha_ref, da_ref, u1a_ref, ma_ref, cab_ref, u2_ref, bu2_ref,
         o_ref) = refs
    u1 = jnp.dot(x_ref[...], u1a_ref[...], preferred_element_type=jnp.float32)
    u1 += jnp.dot(ha_ref[0], ma_ref[0], preferred_element_type=jnp.float32)
    u1 += jnp.dot(ha_ref[1], ma_ref[1], preferred_element_type=jnp.float32)
    u1 += da_ref[...] * cab_ref[0:1, :]
    if has_inc:
        u1 += jnp.dot(hi_ref[0], mi_ref[0], preferred_element_type=jnp.float32)
        u1 += jnp.dot(hi_ref[1], mi_ref[1], preferred_element_type=jnp.float32)
        u1 += di_ref[...] * cab_ref[1:2, :]
    u1 += cab_ref[2:3, :]
    h = _silu(u1)
    o_ref[...] = x_ref[...] + _silu(
        jnp.dot(h, u2_ref[...], preferred_element_type=jnp.float32)
        + bu2_ref[...])


@functools.lru_cache(maxsize=None)
def _upd_call(n, has_inc):
    racc = _racc_for(n)
    specs = [
        pl.BlockSpec((_ROW, _H), lambda i: (i, 0)),
        pl.BlockSpec((2, _ROW, 64), lambda i: (0, i, 0)),
        pl.BlockSpec((_ROW, 1), lambda i: (i, 0)),
    ]
    if has_inc:
        specs += [
            pl.BlockSpec((2, _ROW, 64), lambda i: (0, i, 0)),
            pl.BlockSpec((_ROW, 1), lambda i: (i, 0)),
        ]
    specs += [
        pl.BlockSpec((_H, _H), lambda i: (0, 0)),
        pl.BlockSpec((2, 64, _H), lambda i: (0, 0, 0)),
    ]
    if has_inc:
        specs += [pl.BlockSpec((2, 64, _H), lambda i: (0, 0, 0))]
    specs += [
        pl.BlockSpec((3, _H), lambda i: (0, 0)),
        pl.BlockSpec((_H, _H), lambda i: (0, 0)),
        pl.BlockSpec((1, _H), lambda i: (0, 0)),
    ]
    del racc
    return pl.pallas_call(
        functools.partial(_upd_body, has_inc),
        grid=(n // _ROW,),
        in_specs=specs,
        out_specs=pl.BlockSpec((_ROW, _H), lambda i: (i, 0)),
        out_shape=jax.ShapeDtypeStruct((n, _H), jnp.float32),
    )


def _pool_body(x_ref, b_ref, w1_ref, b1_ref, w2_ref, b2_ref, o_ref):
    h = _silu(jnp.dot(x_ref[...], w1_ref[...],
                      preferred_element_type=jnp.float32) + b1_ref[...])
    h = jnp.dot(h, w2_ref[...], preferred_element_type=jnp.float32) + b2_ref[...]
    gids = lax.broadcasted_iota(jnp.int32, (1, _NG), 1)
    oh = (b_ref[...] == gids).astype(jnp.float32)
    part = lax.dot_general(oh, h, (((0,), (0,)), ((), ())),
                           preferred_element_type=jnp.float32)

    @pl.when(pl.program_id(0) == 0)
    def _():
        o_ref[...] = jnp.zeros_like(o_ref)

    o_ref[...] += part


@functools.lru_cache(maxsize=None)
def _pool_call(n):
    return pl.pallas_call(
        _pool_body,
        grid=(n // _ROW,),
        in_specs=[
            pl.BlockSpec((_ROW, _H), lambda i: (i, 0)),
            pl.BlockSpec((_ROW, 1), lambda i: (i, 0)),
            pl.BlockSpec((_H, _H), lambda i: (0, 0)),
            pl.BlockSpec((1, _H), lambda i: (0, 0)),
            pl.BlockSpec((_H, _H), lambda i: (0, 0)),
            pl.BlockSpec((1, _H), lambda i: (0, 0)),
        ],
        out_specs=pl.BlockSpec((_NG, _H), lambda i: (0, 0)),
        out_shape=jax.ShapeDtypeStruct((_NG, _H), jnp.float32),
    )


def _post_body(s_ref, w1_ref, b1_ref, w2_ref, b2_ref, o_ref):
    h = _silu(jnp.dot(s_ref[...], w1_ref[...],
                      preferred_element_type=jnp.float32) + b1_ref[...])
    o_ref[...] = jnp.dot(h, w2_ref[...],
                         preferred_element_type=jnp.float32) + b2_ref[...]


def _post_call():
    return pl.pallas_call(
        _post_body,
        out_shape=jax.ShapeDtypeStruct((_NG, 1), jnp.float32),
    )


@functools.lru_cache(maxsize=None)
def _edge_sc_call(epad, nacc, na, nb):
    eb = _eb_for(nacc)
    racc = _racc_for(nacc)
    nblk = epad // (_NS * eb)
    assert nblk % 2 == 0 and nblk >= 4
    rows_tile = racc // _NS
    mesh = plsc.VectorSubcoreMesh(core_axis_name="c", subcore_axis_name="s")

    @functools.partial(
        pl.kernel, mesh=mesh,
        compiler_params=pltpu.CompilerParams(use_tc_tiling_on_sc=False),
        out_type=jax.ShapeDtypeStruct((2 * racc, 64), jnp.float32),
        scratch_types=[
            pltpu.VMEM_SHARED((racc, 64), jnp.float32),
            [pltpu.VMEM((3, eb), jnp.int32) for _ in range(2)],
            [pltpu.VMEM((eb,), jnp.int32) for _ in range(2)],
            [pltpu.VMEM((eb,), jnp.int32) for _ in range(2)],
            [pltpu.VMEM((eb,), jnp.int32) for _ in range(2)],
            [pltpu.VMEM((eb, 64), jnp.float32) for _ in range(2)],
            [pltpu.VMEM((eb, 64), jnp.float32) for _ in range(2)],
            [pltpu.VMEM((eb, 64), jnp.float32) for _ in range(2)],
            [pltpu.VMEM((eb, 64), jnp.float32) for _ in range(2)],
            [pltpu.SemaphoreType.DMA for _ in range(2)],
            [pltpu.SemaphoreType.DMA for _ in range(2)],
        ])
    def k(ga, gb, c2, idxall, outh, acc, ib, sg, dg, dsc, ab, bb, cb, ob,
          semg, sems):
        cid = lax.axis_index("c")
        sid = lax.axis_index("s")
        offa = cid * na
        offb = cid * nb

        def zrow(r, _):
            for j in range(4):
                ab[0][r, pl.ds(16 * j, 16)] = jnp.zeros((16,), jnp.float32)
            return 0
        lax.fori_loop(0, eb, zrow, 0)
        r0 = sid * rows_tile
        for q in range(rows_tile // eb):
            pltpu.sync_copy(ab[0], acc.at[pl.ds(r0 + q * eb, eb)])
        plsc.subcore_barrier()

        blk0 = sid * nblk

        def issue(kk, s):
            gblk = blk0 + kk
            pltpu.sync_copy(idxall.at[gblk], ib[s])
            for j in range(eb // 16):
                sl = pl.ds(16 * j, 16)
                sg[s][sl] = ib[s][0, sl] + offa
                dg[s][sl] = ib[s][1, sl] + offb
            pltpu.async_copy(ga.at[sg[s]], ab[s], semg[s])
            pltpu.async_copy(gb.at[dg[s]], bb[s], semg[s])
            pltpu.async_copy(c2.at[pl.ds((cid * epad + gblk * eb), eb)],
                             cb[s], semg[s])

        def finish(kk, s, first):
            pltpu.make_async_copy(ga.at[sg[s]], ab[s], semg[s]).wait()
            pltpu.make_async_copy(gb.at[dg[s]], bb[s], semg[s]).wait()
            pltpu.make_async_copy(c2.at[pl.ds(0, eb)], cb[s], semg[s]).wait()
            if not first:
                pltpu.make_async_copy(ob[s], acc.at[dsc[s]], sems[s]).wait()

            def crow(r, _):
                for j in range(4):
                    sl = pl.ds(16 * j, 16)
                    v = ab[s][r, sl] + bb[s][r, sl] + cb[s][r, sl]
                    ob[s][r, sl] = v / (1.0 + jnp.exp(-v))
                return 0
            lax.fori_loop(0, eb, crow, 0)
            for j in range(eb // 16):
                sl = pl.ds(16 * j, 16)
                dsc[s][sl] = ib[s][2, sl]
            pltpu.async_copy(ob[s], acc.at[dsc[s]], sems[s], add=True)

        issue(0, 0)
        issue(1, 1)

        def pair(p, _):
            kk = 2 * p

            def fin0(first):
                finish(kk, 0, first)
                issue(kk + 2, 0)
                finish(kk + 1, 1, first)
                issue(kk + 3, 1)

            @pl.when(p == 0)
            def _():
                fin0(True)

            @pl.when(p > 0)
            def _():
                fin0(False)
            return 0
        lax.fori_loop(0, nblk // 2 - 1, pair, 0)
        finish(nblk - 2, 0, False)
        finish(nblk - 1, 1, False)
        pltpu.make_async_copy(ob[0], acc.at[dsc[0]], sems[0]).wait()
        pltpu.make_async_copy(ob[1], acc.at[dsc[1]], sems[1]).wait()
        plsc.subcore_barrier()

        pltpu.sync_copy(acc.at[pl.ds(r0, rows_tile)],
                        outh.at[pl.ds(cid * racc + r0, rows_tile)])

    return k


def _edge_sum(ga, gb, c2, idxall, nacc, na, nb):
    racc = _racc_for(nacc)
    eb = _eb_for(nacc)
    epad = idxall.shape[0] * eb
    h = _edge_sc_call(epad, nacc, na, nb)(ga, gb, c2, idxall)
    return h.reshape(2, racc, 64)


def _pad_to(x, m, value):
    e = x.shape[0]
    ep = -(-e // m) * m
    if ep == e:
        return x
    pad = [(0, ep - e)] + [(0, 0)] * (x.ndim - 1)
    return jnp.pad(x, pad, constant_values=value)


def kernel(x0, x1, x2, adj0, adj1, adj2, inc1, inc2, inv_adj0, inv_adj1,
           inv_adj2, inv_inc1, inv_inc2, batch0, batch1, batch2, params):
    xs = [x0, x1, x2]
    sizes = [x.shape[0] for x in xs]
    adjs = [adj0, adj1, adj2]
    incs = [inc1, inc2]
    inva = [inv_adj0, inv_adj1, inv_adj2]
    invi = [inv_inc1, inv_inc2]
    batches = [batch0, batch1, batch2]

    epad = 4096

    def _pack_idx(src, dst, nacc):
        eb = _eb_for(nacc)
        srcp = _pad_to(src.astype(jnp.int32), epad, 0)
        dstg = _pad_to(dst.astype(jnp.int32), epad, 0)
        dsts = _pad_to(dst.astype(jnp.int32), epad, nacc)
        nbt = srcp.shape[0] // eb
        return jnp.stack([srcp.reshape(nbt, eb), dstg.reshape(nbt, eb),
                          dsts.reshape(nbt, eb)], axis=1)

    idx_a = [_pack_idx(adjs[r][0], adjs[r][1], sizes[r]) for r in range(3)]
    idx_i = [_pack_idx(incs[i][0], incs[i][1], sizes[i + 1]) for i in range(2)]

    def _deg(dst, n):
        return jnp.zeros((n,), jnp.float32).at[dst].add(1.0).reshape(n, 1)

    deg_a = [_deg(adjs[r][1], sizes[r]) for r in range(3)]
    deg_i = [_deg(incs[i][1], sizes[i + 1]) for i in range(2)]

    epa = [_pad_to(inva[r], idx_a[r].shape[0] * _eb_for(sizes[r]), 0.0)
           for r in range(3)]
    epi = [_pad_to(invi[i], idx_i[i].shape[0] * _eb_for(sizes[i + 1]), 0.0)
           for i in range(2)]

    xs = [_emb_call(sizes[r])(xs[r], params['emb']['w'],
                              params['emb']['b'][None, :]) for r in range(3)]

    for lp in params['layers']:
        aw = [lp['adj'][r]['l1']['w'] for r in range(3)]
        iw = [lp['inc'][i]['l1']['w'] for i in range(2)]

        def _wcat(ws):
            lo = jnp.concatenate([w[:, :64] for w in ws], axis=1)
            hi = jnp.concatenate([w[:, 64:] for w in ws], axis=1)
            return jnp.stack([lo, hi])

        w0 = _wcat([aw[0][:_H], aw[0][_H:2*_H], iw[0][:_H]])
        w1 = _wcat([aw[1][:_H], aw[1][_H:2*_H], iw[0][_H:2*_H], iw[1][:_H]])
        w2 = _wcat([aw[2][:_H], aw[2][_H:2*_H], iw[1][_H:2*_H]])
        a0A, a0B, i1S = _proj_call(sizes[0], 3)(xs[0], w0)
        a1A, a1B, i1D, i2S = _proj_call(sizes[1], 4)(xs[1], w1)
        a2A, a2B, i2D = _proj_call(sizes[2], 3)(xs[2], w2)

        def _whalf(w):
            return w.reshape(3, 2, 64).transpose(1, 0, 2)

        c_a = [_cprep_call(epa[r].shape[0], epa[r].shape[0])(
                   epa[r], _whalf(aw[r][2*_H:]),
                   lp['adj'][r]['l1']['b'].reshape(2, 1, 64))
               for r in range(3)]
        c_i = [_cprep_call(epi[i].shape[0], epi[i].shape[0])(
                   epi[i], _whalf(iw[i][2*_H:]),
                   lp['inc'][i]['l1']['b'].reshape(2, 1, 64))
               for i in range(2)]

        ed_a = [_edge_sum(pa, pb, c_a[r], idx_a[r], sizes[r],
                          sizes[r], sizes[r])
                for r, (pa, pb) in enumerate([(a0A, a0B), (a1A, a1B),
                                              (a2A, a2B)])]
        ed_i = [_edge_sum(ps, pd, c_i[i], idx_i[i],
                          sizes[i + 1], sizes[i], sizes[i + 1])
                for i, (ps, pd) in enumerate([(i1S, i1D), (i2S, i2D)])]

        new_xs = []
        for r in range(3):
            u = lp['upd'][r]
            U1 = u['l1']['w']
            ma = jnp.stack([lp['adj'][r]['l2']['w'][:64] @ U1[_H:2*_H],
                            lp['adj'][r]['l2']['w'][64:] @ U1[_H:2*_H]])
            ca = lp['adj'][r]['l2']['b'] @ U1[_H:2*_H]
            ha, dega = ed_a[r], deg_a[r]
            if r > 0:
                mi = jnp.stack([lp['inc'][r-1]['l2']['w'][:64] @ U1[2*_H:],
                                lp['inc'][r-1]['l2']['w'][64:] @ U1[2*_H:]])
                ci = lp['inc'][r-1]['l2']['b'] @ U1[2*_H:]
                hi, degi = ed_i[r - 1], deg_i[r - 1]
                cab = jnp.stack([ca, ci, u['l1']['b']])
                new_xs.append(_upd_call(sizes[r], True)(
                    xs[r], ha, dega, hi, degi, U1[:_H], ma, mi, cab,
                    u['l2']['w'], u['l2']['b'][None, :]))
            else:
                cab = jnp.stack([ca, jnp.zeros_like(ca), u['l1']['b']])
                new_xs.append(_upd_call(sizes[r], False)(
                    xs[r], ha, dega, U1[:_H], ma, cab,
                    u['l2']['w'], u['l2']['b'][None, :]))
        xs = new_xs

    pooled = []
    for r in range(3):
        p = params['pre'][r]
        pooled.append(_pool_call(sizes[r])(
            xs[r], batches[r][:, None].astype(jnp.int32),
            p['l1']['w'], p['l1']['b'][None, :],
            p['l2']['w'], p['l2']['b'][None, :]))
    state = jnp.concatenate(pooled, axis=1)
    p = params['post']
    out = _post_call()(state, p['l1']['w'], p['l1']['b'][None, :],
                       p['l2']['w'], p['l2']['b'][None, :])
    return out[:, 0]
